# baseline (speedup 1.0000x reference)
import dataclasses
import functools

import jax
import jax.numpy as jnp
from jax import lax
from jax.experimental import pallas as pl
from jax.experimental.pallas import tpu as pltpu
from jax.experimental.pallas import tpu_sc as plsc

N = 10000
E = 320000
D = 128
DH = D // 2

NC, NS, L = 2, 16, 16
NW = NC * NS
NP = 10240
RPT = NP // NS
CHUNK = 128
EPT = E // NS
CPT = 160
EPT_PAD = CPT * CHUNK
CPW = CPT // NC
PAD_NODE = N
NPS = 10112
RPTS = NPS // NS
SUB = 64
NB = 5

_mesh = functools.partial(
    plsc.VectorSubcoreMesh, core_axis_name="c", subcore_axis_name="s",
    num_cores=NC, num_subcores=NS)


def _sc_compiler_params(tc_tiling=True):
    cp = pltpu.CompilerParams()
    if "needs_layout_passes" in pltpu.CompilerParams.__dataclass_fields__:
        cp = dataclasses.replace(cp, needs_layout_passes=False)
    if not tc_tiling:
        cp = dataclasses.replace(cp, use_tc_tiling_on_sc=False)
    return cp


def _hist_body(idx_hbm, out_hbm, idx_v, hist_s, hist_d):
    cid = lax.axis_index("c")
    sid = lax.axis_index("s")
    wid = cid * NS + sid

    @pl.loop(0, NP, step=L)
    def _(i):
        z = jnp.zeros((L,), jnp.float32)
        hist_s[pl.ds(i, L)] = z
        hist_d[pl.ds(i, L)] = z

    pltpu.sync_copy(idx_hbm.at[0, sid, pl.ds(cid * CPW, CPW)], idx_v.at[0])
    pltpu.sync_copy(idx_hbm.at[1, sid, pl.ds(cid * CPW, CPW)], idx_v.at[1])

    ones = jnp.ones((L,), jnp.float32)

    @pl.loop(0, CPW)
    def _(j):
        for t in range(2):
            @pl.loop(0, SUB, step=L)
            def _(i):
                plsc.addupdate_scatter(
                    hist_s, [idx_v[0, j, t, pl.ds(i, L)]], ones)
                plsc.addupdate_scatter(
                    hist_d, [idx_v[1, j, t, pl.ds(i, L)]], ones)

    pltpu.sync_copy(hist_s, out_hbm.at[wid, 0])
    pltpu.sync_copy(hist_d, out_hbm.at[wid, 1])


def _sc_hist(idx_all):
    kern = pl.kernel(
        _hist_body,
        out_type=jax.ShapeDtypeStruct((NW, 2, NP), jnp.float32),
        mesh=_mesh(),
        scratch_types=[
            pltpu.VMEM((2, CPW, 2, SUB), jnp.int32),
            pltpu.VMEM((NP,), jnp.float32),
            pltpu.VMEM((NP,), jnp.float32),
        ],
        compiler_params=_sc_compiler_params(),
    )
    return kern(idx_all)


def _agg_body(p_hbm, src_hbm, dst_hbm, out_hbm, srcv, dstv,
              gb0, gb1, p_sh, agg_sh, *sems):
    cid = lax.axis_index("c")
    sid = lax.axis_index("s")

    @pl.loop(0, SUB)
    def _(r):
        @pl.loop(0, DH, step=L)
        def _(c):
            gb0[r, pl.ds(c, L)] = jnp.zeros((L,), jnp.float32)

    @pl.loop(0, RPTS - RPTS % SUB, step=SUB)
    def _(r):
        pltpu.sync_copy(gb0, agg_sh.at[pl.ds(sid * RPTS + r, SUB)])

    tail = RPTS % SUB
    pltpu.sync_copy(gb0.at[pl.ds(0, tail)],
                    agg_sh.at[pl.ds(sid * RPTS + (RPTS - tail), tail)])

    pltpu.sync_copy(p_hbm.at[cid, pl.ds(sid * RPTS, RPTS)],
                    p_sh.at[pl.ds(sid * RPTS, RPTS)])
    pltpu.sync_copy(src_hbm.at[sid], srcv)
    pltpu.sync_copy(dst_hbm.at[sid], dstv)
    plsc.subcore_barrier()

    @pl.loop(0, CPT)
    def _(j):
        pltpu.async_copy(p_sh.at[srcv.at[j, 1]], gb1, sems[0])
        pltpu.sync_copy(p_sh.at[srcv.at[j, 0]], gb0)
        pltpu.sync_copy(gb0, agg_sh.at[dstv.at[j, 0]], add=True)
        pltpu.make_async_copy(p_sh.at[srcv.at[j, 1]], gb1, sems[0]).wait()
        pltpu.sync_copy(gb1, agg_sh.at[dstv.at[j, 1]], add=True)

    plsc.subcore_barrier()
    pltpu.sync_copy(agg_sh.at[pl.ds(sid * RPTS, RPTS)],
                    out_hbm.at[cid, pl.ds(sid * RPTS, RPTS)])


def _sc_aggregate(p_halves, src_p, dst_p):
    kern = pl.kernel(
        _agg_body,
        out_type=jax.ShapeDtypeStruct((NC, NP, DH), jnp.float32),
        mesh=_mesh(),
        scratch_types=[
            pltpu.VMEM((CPT, 2, SUB), jnp.int32),
            pltpu.VMEM((CPT, 2, SUB), jnp.int32),
        ] + [pltpu.VMEM((SUB, DH), jnp.float32)] * 2 + [
            pltpu.VMEM_SHARED((NPS, DH), jnp.float32),
            pltpu.VMEM_SHARED((NPS, DH), jnp.float32),
        ] + [pltpu.SemaphoreType.DMA],
        compiler_params=_sc_compiler_params(tc_tiling=False),
    )
    return kern(p_halves, src_p, dst_p)


_RB = 512


def _mm_scale_body(x_ref, w_ref, deg_ref, o_ref):
    ns = lax.rsqrt(jnp.maximum(
        jnp.sum(deg_ref[:, 0:NW], axis=1, keepdims=True), 1.0))
    acc = jnp.dot(x_ref[...], w_ref[...],
                  preferred_element_type=jnp.float32,
                  precision=lax.Precision.HIGHEST)
    acc = acc * ns
    o_ref[0] = acc[:, :DH]
    o_ref[1] = acc[:, DH:]


def _tc_mm_scale(x, w, degs):
    return pl.pallas_call(
        _mm_scale_body,
        grid=(NP // _RB,),
        in_specs=[
            pl.BlockSpec((_RB, D), lambda i: (i, 0)),
            pl.BlockSpec((D, D), lambda i: (0, 0)),
            pl.BlockSpec((_RB, 2 * NW), lambda i: (i, 0)),
        ],
        out_specs=pl.BlockSpec((NC, _RB, DH), lambda i: (0, i, 0)),
        out_shape=jax.ShapeDtypeStruct((NC, NP, DH), jnp.float32),
    )(x, w, degs)


def _mid_body(a_ref, deg_ref, b_ref, w_ref, o_ref):
    ns = lax.rsqrt(jnp.maximum(
        jnp.sum(deg_ref[:, 0:NW], axis=1, keepdims=True), 1.0))
    nd = lax.rsqrt(jnp.maximum(
        jnp.sum(deg_ref[:, NW:2 * NW], axis=1, keepdims=True), 1.0))
    agg = jnp.concatenate([a_ref[0], a_ref[1]], axis=1)
    h = agg * nd + b_ref[...]
    h = jnp.maximum(h, 0.0)
    acc = jnp.dot(h, w_ref[...], preferred_element_type=jnp.float32,
                  precision=lax.Precision.HIGHEST)
    acc = acc * ns
    o_ref[0] = acc[:, :DH]
    o_ref[1] = acc[:, DH:]


def _tc_mid(a, degs, b1, w2):
    return pl.pallas_call(
        _mid_body,
        grid=(NP // _RB,),
        in_specs=[
            pl.BlockSpec((NC, _RB, DH), lambda i: (0, i, 0)),
            pl.BlockSpec((_RB, 2 * NW), lambda i: (i, 0)),
            pl.BlockSpec((1, D), lambda i: (0, 0)),
            pl.BlockSpec((D, D), lambda i: (0, 0)),
        ],
        out_specs=pl.BlockSpec((NC, _RB, DH), lambda i: (0, i, 0)),
        out_shape=jax.ShapeDtypeStruct((NC, NP, DH), jnp.float32),
    )(a, degs, b1, w2)


_RBF = 400


def _final_body(a_ref, deg_ref, b_ref, o_ref):
    nd = lax.rsqrt(jnp.maximum(
        jnp.sum(deg_ref[:, NW:2 * NW], axis=1, keepdims=True), 1.0))
    agg = jnp.concatenate([a_ref[0], a_ref[1]], axis=1)
    o_ref[...] = agg * nd + b_ref[...]


def _tc_final(a, degs, b2):
    return pl.pallas_call(
        _final_body,
        grid=(N // _RBF,),
        in_specs=[
            pl.BlockSpec((NC, _RBF, DH), lambda i: (0, i, 0)),
            pl.BlockSpec((_RBF, 2 * NW), lambda i: (i, 0)),
            pl.BlockSpec((1, D), lambda i: (0, 0)),
        ],
        out_specs=pl.BlockSpec((_RBF, D), lambda i: (i, 0)),
        out_shape=jax.ShapeDtypeStruct((N, D), jnp.float32),
    )(a, degs, b2)


def kernel(in_feat, edge_index, W1, b1, W2, b2):
    src = edge_index[0].astype(jnp.int32)
    dst = edge_index[1].astype(jnp.int32)

    def pad_idx(a):
        a = a.reshape(NS, EPT)
        a = jnp.pad(a, ((0, 0), (0, EPT_PAD - EPT)),
                    constant_values=PAD_NODE)
        return a.reshape(NS, CPT, 2, SUB)

    src_p = pad_idx(src)
    dst_p = pad_idx(dst)
    idx_all = jnp.stack([src_p, dst_p])

    degs = _sc_hist(idx_all)
    degs8 = jnp.transpose(degs, (2, 1, 0)).reshape(NP, 2 * NW)

    x_pad = jnp.pad(in_feat, ((0, NP - N), (0, 0)))

    p1 = _tc_mm_scale(x_pad, W1, degs8)
    agg1 = _sc_aggregate(p1, src_p, dst_p)
    p2 = _tc_mid(agg1, degs8, b1.reshape(1, D), W2)
    agg2 = _sc_aggregate(p2, src_p, dst_p)
    out = _tc_final(agg2, degs8, b2.reshape(1, D))
    return out


# speedup vs baseline: 1.0310x; 1.0310x over previous
import dataclasses
import functools

import jax
import jax.numpy as jnp
from jax import lax
from jax.experimental import pallas as pl
from jax.experimental.pallas import tpu as pltpu
from jax.experimental.pallas import tpu_sc as plsc

N = 10000
E = 320000
D = 128
DH = D // 2

NC, NS, L = 2, 16, 16
NW = NC * NS
NP = 10240
NPS = 10112
RPTS = NPS // NS
CHUNK = 128
EPT = E // NS
CPT = 160
EPT_PAD = CPT * CHUNK
CPW = CPT // NC
PAD_NODE = N

_mesh = functools.partial(
    plsc.VectorSubcoreMesh, core_axis_name="c", subcore_axis_name="s",
    num_cores=NC, num_subcores=NS)


def _sc_compiler_params(tc_tiling=True):
    cp = pltpu.CompilerParams()
    if "needs_layout_passes" in pltpu.CompilerParams.__dataclass_fields__:
        cp = dataclasses.replace(cp, needs_layout_passes=False)
    if not tc_tiling:
        cp = dataclasses.replace(cp, use_tc_tiling_on_sc=False)
    return cp


def _hist_body(src_hbm, dst_hbm, out_hbm, idx_v, hist_s, hist_d):
    cid = lax.axis_index("c")
    sid = lax.axis_index("s")
    wid = cid * NS + sid

    @pl.loop(0, NP, step=L)
    def _(i):
        z = jnp.zeros((L,), jnp.float32)
        hist_s[pl.ds(i, L)] = z
        hist_d[pl.ds(i, L)] = z

    pltpu.sync_copy(src_hbm.at[sid, pl.ds(cid * CPW, CPW)], idx_v.at[0])
    pltpu.sync_copy(dst_hbm.at[sid, pl.ds(cid * CPW, CPW)], idx_v.at[1])

    ones = jnp.ones((L,), jnp.float32)

    @pl.loop(0, CPW)
    def _(j):
        @pl.loop(0, CHUNK, step=L)
        def _(i):
            plsc.addupdate_scatter(hist_s, [idx_v[0, j, pl.ds(i, L)]], ones)
            plsc.addupdate_scatter(hist_d, [idx_v[1, j, pl.ds(i, L)]], ones)

    pltpu.sync_copy(hist_s, out_hbm.at[wid, 0])
    pltpu.sync_copy(hist_d, out_hbm.at[wid, 1])


def _sc_hist(src_p, dst_p):
    kern = pl.kernel(
        _hist_body,
        out_type=jax.ShapeDtypeStruct((NW, 2, NP), jnp.float32),
        mesh=_mesh(),
        scratch_types=[
            pltpu.VMEM((2, CPW, CHUNK), jnp.int32),
            pltpu.VMEM((NP,), jnp.float32),
            pltpu.VMEM((NP,), jnp.float32),
        ],
        compiler_params=_sc_compiler_params(),
    )
    return kern(src_p, dst_p)


def _agg_body(p_hbm, src_hbm, dst_hbm, out_hbm, srcv, dstv,
              gb0, p_sh, agg_sh):
    cid = lax.axis_index("c")
    sid = lax.axis_index("s")

    @pl.loop(0, CHUNK)
    def _(r):
        @pl.loop(0, DH, step=L)
        def _(c):
            gb0[r, pl.ds(c, L)] = jnp.zeros((L,), jnp.float32)

    @pl.loop(0, RPTS - RPTS % CHUNK, step=CHUNK)
    def _(r):
        pltpu.sync_copy(gb0, agg_sh.at[pl.ds(sid * RPTS + r, CHUNK)])

    tail = RPTS % CHUNK
    pltpu.sync_copy(gb0.at[pl.ds(0, tail)],
                    agg_sh.at[pl.ds(sid * RPTS + (RPTS - tail), tail)])

    pltpu.sync_copy(p_hbm.at[cid, pl.ds(sid * RPTS, RPTS)],
                    p_sh.at[pl.ds(sid * RPTS, RPTS)])
    pltpu.sync_copy(src_hbm.at[sid], srcv)
    pltpu.sync_copy(dst_hbm.at[sid], dstv)
    plsc.subcore_barrier()

    @pl.loop(0, CPT)
    def _(j):
        pltpu.sync_copy(p_sh.at[srcv.at[j]], gb0)
        pltpu.sync_copy(gb0, agg_sh.at[dstv.at[j]], add=True)

    plsc.subcore_barrier()
    pltpu.sync_copy(agg_sh.at[pl.ds(sid * RPTS, RPTS)],
                    out_hbm.at[cid, pl.ds(sid * RPTS, RPTS)])


def _sc_aggregate(p_halves, src_p, dst_p):
    kern = pl.kernel(
        _agg_body,
        out_type=jax.ShapeDtypeStruct((NC, NP, DH), jnp.float32),
        mesh=_mesh(),
        scratch_types=[
            pltpu.VMEM((CPT, CHUNK), jnp.int32),
            pltpu.VMEM((CPT, CHUNK), jnp.int32),
            pltpu.VMEM((CHUNK, DH), jnp.float32),
            pltpu.VMEM_SHARED((NPS, DH), jnp.float32),
            pltpu.VMEM_SHARED((NPS, DH), jnp.float32),
        ],
        compiler_params=_sc_compiler_params(tc_tiling=False),
    )
    return kern(p_halves, src_p, dst_p)


_RB = 400


def _mm_scale_body(x_ref, w_ref, deg_ref, o_ref):
    ns = lax.rsqrt(jnp.maximum(
        jnp.sum(deg_ref[:, 0:NW], axis=1, keepdims=True), 1.0))
    acc = jnp.dot(x_ref[...], w_ref[...],
                  preferred_element_type=jnp.float32,
                  precision=lax.Precision.HIGHEST)
    acc = acc * ns
    o_ref[0] = acc[:, :DH]
    o_ref[1] = acc[:, DH:]


def _tc_mm_scale(x, w, degs):
    return pl.pallas_call(
        _mm_scale_body,
        grid=(N // _RB,),
        in_specs=[
            pl.BlockSpec((_RB, D), lambda i: (i, 0)),
            pl.BlockSpec((D, D), lambda i: (0, 0)),
            pl.BlockSpec((_RB, 2 * NW), lambda i: (i, 0)),
        ],
        out_specs=pl.BlockSpec((NC, _RB, DH), lambda i: (0, i, 0)),
        out_shape=jax.ShapeDtypeStruct((NC, NP, DH), jnp.float32),
    )(x, w, degs)


def _mid_body(a_ref, deg_ref, b_ref, w_ref, o_ref):
    ns = lax.rsqrt(jnp.maximum(
        jnp.sum(deg_ref[:, 0:NW], axis=1, keepdims=True), 1.0))
    nd = lax.rsqrt(jnp.maximum(
        jnp.sum(deg_ref[:, NW:2 * NW], axis=1, keepdims=True), 1.0))
    agg = jnp.concatenate([a_ref[0], a_ref[1]], axis=1)
    h = agg * nd + b_ref[...]
    h = jnp.maximum(h, 0.0)
    acc = jnp.dot(h, w_ref[...], preferred_element_type=jnp.float32,
                  precision=lax.Precision.HIGHEST)
    acc = acc * ns
    o_ref[0] = acc[:, :DH]
    o_ref[1] = acc[:, DH:]


def _tc_mid(a, degs, b1, w2):
    return pl.pallas_call(
        _mid_body,
        grid=(N // _RB,),
        in_specs=[
            pl.BlockSpec((NC, _RB, DH), lambda i: (0, i, 0)),
            pl.BlockSpec((_RB, 2 * NW), lambda i: (i, 0)),
            pl.BlockSpec((1, D), lambda i: (0, 0)),
            pl.BlockSpec((D, D), lambda i: (0, 0)),
        ],
        out_specs=pl.BlockSpec((NC, _RB, DH), lambda i: (0, i, 0)),
        out_shape=jax.ShapeDtypeStruct((NC, NP, DH), jnp.float32),
    )(a, degs, b1, w2)


def _final_body(a_ref, deg_ref, b_ref, o_ref):
    nd = lax.rsqrt(jnp.maximum(
        jnp.sum(deg_ref[:, NW:2 * NW], axis=1, keepdims=True), 1.0))
    agg = jnp.concatenate([a_ref[0], a_ref[1]], axis=1)
    o_ref[...] = agg * nd + b_ref[...]


def _tc_final(a, degs, b2):
    return pl.pallas_call(
        _final_body,
        grid=(N // _RB,),
        in_specs=[
            pl.BlockSpec((NC, _RB, DH), lambda i: (0, i, 0)),
            pl.BlockSpec((_RB, 2 * NW), lambda i: (i, 0)),
            pl.BlockSpec((1, D), lambda i: (0, 0)),
        ],
        out_specs=pl.BlockSpec((_RB, D), lambda i: (i, 0)),
        out_shape=jax.ShapeDtypeStruct((N, D), jnp.float32),
    )(a, degs, b2)


def kernel(in_feat, edge_index, W1, b1, W2, b2):
    src = edge_index[0].astype(jnp.int32)
    dst = edge_index[1].astype(jnp.int32)

    def pad_idx(a):
        a = a.reshape(NS, EPT)
        a = jnp.pad(a, ((0, 0), (0, EPT_PAD - EPT)),
                    constant_values=PAD_NODE)
        return a.reshape(NS, CPT, CHUNK)

    src_p = pad_idx(src)
    dst_p = pad_idx(dst)

    degs = _sc_hist(src_p, dst_p)
    degs8 = jnp.transpose(degs, (2, 1, 0)).reshape(NP, 2 * NW)

    p1 = _tc_mm_scale(in_feat, W1, degs8)
    agg1 = _sc_aggregate(p1, src_p, dst_p)
    p2 = _tc_mid(agg1, degs8, b1.reshape(1, D), W2)
    agg2 = _sc_aggregate(p2, src_p, dst_p)
    out = _tc_final(agg2, degs8, b2.reshape(1, D))
    return out
